# baseline (speedup 1.0000x reference)
import functools

import jax
import jax.numpy as jnp
from jax import lax
from jax.experimental import pallas as pl
from jax.experimental.pallas import tpu as pltpu
from jax.experimental.pallas import tpu_sc as plsc

_BATCH = 4
_NC = 2
_NS = 16
_NW = _NC * _NS


def kernel(encoding, batch_size, seq_len):
    max_len, dim = encoding.shape
    rows_per_w = max_len // _NW

    mesh = plsc.VectorSubcoreMesh(core_axis_name="c", subcore_axis_name="s")

    @functools.partial(
        pl.kernel,
        mesh=mesh,
        out_type=jax.ShapeDtypeStruct((_BATCH, max_len, dim), encoding.dtype),
        scratch_types=[pltpu.SemaphoreType.DMA],
    )
    def sc_copy(enc_hbm, out_hbm, sem):
        wid = lax.axis_index("s") * _NC + lax.axis_index("c")
        base = wid * rows_per_w
        src = enc_hbm.at[pl.ds(base, rows_per_w)]
        copies = [
            pltpu.make_async_copy(src, out_hbm.at[b, pl.ds(base, rows_per_w)], sem)
            for b in range(_BATCH)
        ]
        for c in copies:
            c.start()
        for c in copies:
            c.wait()

    return sc_copy(encoding)


# speedup vs baseline: 55.1709x; 55.1709x over previous
import functools

import jax
import jax.numpy as jnp
from jax import lax
from jax.experimental import pallas as pl
from jax.experimental.pallas import tpu as pltpu
from jax.experimental.pallas import tpu_sc as plsc

_BATCH = 4
_NC = 2
_NS = 16
_NW = _NC * _NS


_CHUNK = 32
_NBUF = 2


def kernel(encoding, batch_size, seq_len):
    max_len, dim = encoding.shape
    rows_per_w = max_len // _NW
    n_chunks = rows_per_w // _CHUNK

    mesh = plsc.VectorSubcoreMesh(core_axis_name="c", subcore_axis_name="s")

    @functools.partial(
        pl.kernel,
        mesh=mesh,
        out_type=jax.ShapeDtypeStruct((_BATCH, max_len, dim), encoding.dtype),
        scratch_types=[
            pltpu.VMEM((_NBUF, _CHUNK, dim), jnp.float32),
            pltpu.SemaphoreType.DMA,
            pltpu.SemaphoreType.DMA,
        ],
    )
    def sc_copy(enc_hbm, out_hbm, bufs, in_sem, out_sem):
        wid = lax.axis_index("s") * _NC + lax.axis_index("c")
        base = wid * rows_per_w

        def in_copy(i, slot):
            return pltpu.make_async_copy(
                enc_hbm.at[pl.ds(base + i * _CHUNK, _CHUNK)],
                bufs.at[slot],
                in_sem,
            )

        def out_copies(i, slot):
            return [
                pltpu.make_async_copy(
                    bufs.at[slot],
                    out_hbm.at[b, pl.ds(base + i * _CHUNK, _CHUNK)],
                    out_sem,
                )
                for b in range(_BATCH)
            ]

        in_copy(0, 0).start()

        def step(i, _):
            slot = lax.rem(i, _NBUF)
            in_copy(i, slot).wait()

            @pl.when(i + 1 < n_chunks)
            def _():
                in_copy(i + 1, lax.rem(i + 1, _NBUF)).start()

            cs = out_copies(i, slot)
            for c in cs:
                c.start()
            for c in cs:
                c.wait()
            return 0

        lax.fori_loop(0, n_chunks, step, 0)

    return sc_copy(encoding)
